# Initial kernel scaffold; baseline (speedup 1.0000x reference)
import jax
import jax.numpy as jnp
from jax.experimental import pallas as pl


def kernel(feature, lan_emb, W_w, b_w, W_out, b_out):
    raise NotImplementedError("write your pallas kernel here")


# speedup vs baseline: 1.5435x; 1.5435x over previous
import functools

import jax
import jax.numpy as jnp
from jax import lax
from jax.experimental import pallas as pl
from jax.experimental.pallas import tpu as pltpu
from jax.experimental.pallas import tpu_sc as plsc

VOCAB = 1000000
D = 64
VB = 8000
NB = VOCAB // VB

NC = 2
NS = 16
NW = NC * NS
CH = 128
N_IDX = 4096 * 200
PER_W = N_IDX // NW
NCHUNK = PER_W // CH
K_FIRE = 8
NGROUP = NCHUNK // K_FIRE


def _proj_body(x_ref, w_ref, b_ref, o_ref):
    x = x_ref[0]
    y = lax.dot_general(w_ref[...], x, (((1,), (1,)), ((), ())),
                        preferred_element_type=jnp.float32)
    o_ref[0] = y + b_ref[0]


def _gather_body(proj_hbm, idx_hbm, out_hbm, idx_v, val_v, sem):
    wid = lax.axis_index("s") * NC + lax.axis_index("c")
    pltpu.sync_copy(idx_hbm.at[wid], idx_v)

    def group(g, carry):
        base = g * K_FIRE
        copies = [
            pltpu.async_copy(proj_hbm.at[idx_v.at[base + k]], val_v.at[base + k], sem)
            for k in range(K_FIRE)
        ]
        for c in copies:
            c.wait()
        return carry

    lax.fori_loop(0, NGROUP, group, 0)
    pltpu.sync_copy(val_v, out_hbm.at[wid])


def kernel(feature, lan_emb, W_w, b_w, W_out, b_out):
    table3 = lan_emb.reshape(NB, VB, D)
    proj3 = pl.pallas_call(
        _proj_body,
        grid=(NB,),
        in_specs=[
            pl.BlockSpec((1, VB, D), lambda i: (i, 0, 0)),
            pl.BlockSpec((1, D), lambda i: (0, 0)),
            pl.BlockSpec(memory_space=pltpu.SMEM),
        ],
        out_specs=pl.BlockSpec((1, 1, VB), lambda i: (i, 0, 0)),
        out_shape=jax.ShapeDtypeStruct((NB, 1, VB), jnp.float32),
    )(table3, W_out, b_out)
    proj = proj3.reshape(VOCAB)

    idx3 = feature.astype(jnp.int32).reshape(NW, NCHUNK, CH)

    gather = functools.partial(
        pl.kernel,
        mesh=plsc.VectorSubcoreMesh(core_axis_name="c", subcore_axis_name="s"),
        out_type=jax.ShapeDtypeStruct((NW, NCHUNK, CH), jnp.float32),
        scratch_types=[
            pltpu.VMEM((NCHUNK, CH), jnp.int32),
            pltpu.VMEM((NCHUNK, CH), jnp.float32),
            pltpu.SemaphoreType.DMA,
        ],
    )(_gather_body)
    out3 = gather(proj, idx3)

    return out3.reshape(4096, 200)
